# Initial kernel scaffold; baseline (speedup 1.0000x reference)
import jax
import jax.numpy as jnp
from jax.experimental import pallas as pl


def kernel(centers, contexts, negatives, W_in, W_out):
    raise NotImplementedError("write your pallas kernel here")


# speedup vs baseline: 1.7070x; 1.7070x over previous
import functools

import jax
import jax.numpy as jnp
from jax import lax
from jax.experimental import pallas as pl
from jax.experimental.pallas import tpu as pltpu
from jax.experimental.pallas import tpu_sc as plsc

VOCAB_ = 100000
D_ = 128
B_ = 16384
K_ = 20
NW_ = 32
BPW_ = B_ // NW_
CH_ = 16
NCHUNK_ = BPW_ // CH_
SEGS_ = ((0, 16), (16, 32), (32, 64), (64, 128))


def _iota16():
    return lax.iota(jnp.int32, 16)


def _sc_body(ctr_hbm, ctx_hbm, neg_hbm, win_hbm, wout_hbm,
             outa_hbm, outb_hbm, outc_hbm,
             ctr_v, ctx_v, negidx_v, negrows_v, vc_v, vp_v,
             outa_v, outb_v, outc_v, sem):
    nc = plsc.get_sparse_core_info().num_cores
    wid = lax.axis_index("s") * nc + lax.axis_index("c")

    def chunk(c, _):
        b0 = pl.multiple_of(wid * BPW_ + c * CH_, CH_)
        pltpu.sync_copy(ctr_hbm.at[pl.ds(b0, CH_)], ctr_v)
        pltpu.sync_copy(ctx_hbm.at[pl.ds(b0, CH_)], ctx_v)
        pltpu.sync_copy(neg_hbm.at[pl.ds(b0 * K_, CH_ * K_)], negidx_v)
        waits = []
        for j in range(4):
            sl = pl.ds(j * 80, 80)
            waits.append(pltpu.async_copy(
                wout_hbm.at[negidx_v.at[sl]], negrows_v.at[sl], sem))
        waits.append(pltpu.async_copy(win_hbm.at[ctr_v], vc_v, sem))
        waits.append(pltpu.async_copy(wout_hbm.at[ctx_v], vp_v, sem))
        for h in waits:
            h.wait()

        lanes = _iota16()

        def pass1(b, _):
            rows = b * K_ + lanes

            def blk_step(blk, acc):
                vcb = vc_v[b, pl.ds(blk * 16, 16)]
                for j in range(16):
                    cv = vcb.at[jnp.full((16,), j, jnp.int32)].get(
                        mode="promise_in_bounds")
                    cols = jnp.full((16,), 0, jnp.int32) + (blk * 16 + j)
                    vals = plsc.load_gather(negrows_v, [rows, cols])
                    acc = acc + vals * cv
                return acc

            acc = jnp.zeros((16,), jnp.float32)
            for mi, (lo, hi) in enumerate(SEGS_):
                acc = lax.fori_loop(lo // 16, hi // 16, blk_step, acc)
                outa_v[b, mi, :] = acc
            return 0

        lax.fori_loop(0, CH_, pass1, 0)

        def pass2(sub, _):
            bvec = sub * 4 + lanes // 4
            rows = bvec * K_ + 16 + (lanes % 4)

            def dstep(d, acc):
                cols = jnp.full((16,), 0, jnp.int32) + d
                vals = plsc.load_gather(negrows_v, [rows, cols])
                cv = plsc.load_gather(vc_v, [bvec, cols])
                return acc + vals * cv

            acc = jnp.zeros((16,), jnp.float32)
            for mi, (lo, hi) in enumerate(SEGS_):
                acc = lax.fori_loop(lo, hi, dstep, acc, unroll=8)
                outb_v[sub, mi, :] = acc
            return 0

        lax.fori_loop(0, 4, pass2, 0)

        def dstep3(d, acc):
            cols = jnp.full((16,), 0, jnp.int32) + d
            pv = plsc.load_gather(vp_v, [lanes, cols])
            cv = plsc.load_gather(vc_v, [lanes, cols])
            return acc + pv * cv

        acc = jnp.zeros((16,), jnp.float32)
        for mi, (lo, hi) in enumerate(SEGS_):
            acc = lax.fori_loop(lo, hi, dstep3, acc, unroll=8)
            outc_v[0, mi, :] = acc

        pltpu.sync_copy(outa_v, outa_hbm.at[pl.ds(b0, CH_)])
        pltpu.sync_copy(outb_v, outb_hbm.at[pl.ds(b0 // 4, 4)])
        pltpu.sync_copy(outc_v, outc_hbm.at[pl.ds(b0 // CH_, 1)])
        return 0

    lax.fori_loop(0, NCHUNK_, chunk, 0)


def _sc_scores(centers, contexts, negflat, w_in, w_out):
    mesh = plsc.VectorSubcoreMesh(core_axis_name="c", subcore_axis_name="s")
    f32 = jnp.float32
    kern = functools.partial(
        pl.kernel,
        out_type=(
            jax.ShapeDtypeStruct((B_, 4, 16), f32),
            jax.ShapeDtypeStruct((B_ // 4, 4, 16), f32),
            jax.ShapeDtypeStruct((B_ // CH_, 4, 16), f32),
        ),
        mesh=mesh,
        compiler_params=pltpu.CompilerParams(needs_layout_passes=False),
        scratch_types=[
            pltpu.VMEM((CH_,), jnp.int32),
            pltpu.VMEM((CH_,), jnp.int32),
            pltpu.VMEM((CH_ * K_,), jnp.int32),
            pltpu.VMEM((CH_ * K_, D_), f32),
            pltpu.VMEM((CH_, D_), f32),
            pltpu.VMEM((CH_, D_), f32),
            pltpu.VMEM((CH_, 4, 16), f32),
            pltpu.VMEM((4, 4, 16), f32),
            pltpu.VMEM((1, 4, 16), f32),
            pltpu.SemaphoreType.DMA,
        ],
    )(_sc_body)
    return kern(centers, contexts, negflat, w_in, w_out)


def _tc_reduce(nega, negb, posc):
    def body(a_ref, b_ref, c_ref, o_ref):
        s = jnp.sum(jax.nn.log_sigmoid(-a_ref[...]))
        s = s + jnp.sum(jax.nn.log_sigmoid(-b_ref[...]))
        s = s + jnp.sum(jax.nn.log_sigmoid(c_ref[...]))
        o_ref[...] = jnp.broadcast_to(-s * (0.25 / B_), (1, 1))

    return pl.pallas_call(
        body,
        out_shape=jax.ShapeDtypeStruct((1, 1), jnp.float32),
    )(nega, negb, posc)


def kernel(centers, contexts, negatives, W_in, W_out):
    centers = centers.astype(jnp.int32)
    contexts = contexts.astype(jnp.int32)
    negflat = negatives.astype(jnp.int32).reshape(B_ * K_)
    nega, negb, posc = _sc_scores(centers, contexts, negflat, W_in, W_out)
    loss = _tc_reduce(
        nega.reshape(B_ * 64 // 128, 128),
        negb.reshape(B_ * 16 // 128, 128),
        posc.reshape(B_ * 4 // 128, 128),
    )
    return loss.reshape(())


# speedup vs baseline: 1.9847x; 1.1626x over previous
import functools

import jax
import jax.numpy as jnp
from jax import lax
from jax.experimental import pallas as pl
from jax.experimental.pallas import tpu as pltpu
from jax.experimental.pallas import tpu_sc as plsc

VOCAB_ = 100000
D_ = 128
B_ = 16384
K_ = 20
NW_ = 32
BPW_ = B_ // NW_
CH_ = 16
NCHUNK_ = BPW_ // CH_
SEGS_ = ((0, 16), (16, 32), (32, 64), (64, 128))


def _i32x16(v):
    return jnp.zeros((16,), jnp.int32) + v


def _sc_body(ctr_hbm, ctx_hbm, neg_hbm, win_hbm, wout_hbm,
             outa_hbm, outb_hbm, outc_hbm,
             ctr0, ctx0, neg0, rows0, vc0, vp0,
             ctr1, ctx1, neg1, rows1, vc1, vp1,
             oa0, ob0, oc0, oa1, ob1, oc1, sems):
    nc = plsc.get_sparse_core_info().num_cores
    wid = lax.axis_index("s") * nc + lax.axis_index("c")
    lanes = lax.iota(jnp.int32, 16)
    rsem0, rsem1 = sems.at[0], sems.at[1]
    isem0, isem1 = sems.at[2], sems.at[3]
    osem0, osem1 = sems.at[4], sems.at[5]

    def base_of(c):
        return pl.multiple_of(wid * BPW_ + c * CH_, CH_)

    def stage_idx(c, ctr_v, ctx_v, neg_v, isem):
        b0 = base_of(c)
        pltpu.make_async_copy(ctr_hbm.at[pl.ds(b0, CH_)], ctr_v, isem).start()
        pltpu.make_async_copy(ctx_hbm.at[pl.ds(b0, CH_)], ctx_v, isem).start()
        pltpu.make_async_copy(
            neg_hbm.at[pl.ds(b0 * K_, CH_ * K_)], neg_v, isem).start()

    def wait_idx(ctr_v, ctx_v, neg_v, isem):
        pltpu.make_async_copy(ctr_hbm.at[pl.ds(0, CH_)], ctr_v, isem).wait()
        pltpu.make_async_copy(ctx_hbm.at[pl.ds(0, CH_)], ctx_v, isem).wait()
        pltpu.make_async_copy(
            neg_hbm.at[pl.ds(0, CH_ * K_)], neg_v, isem).wait()

    def fire_rows(ctr_v, ctx_v, neg_v, rows_v, vc_v, vp_v, rsem):
        for j in range(4):
            sl = pl.ds(j * 80, 80)
            pltpu.make_async_copy(
                wout_hbm.at[neg_v.at[sl]], rows_v.at[sl], rsem).start()
        pltpu.make_async_copy(win_hbm.at[ctr_v], vc_v, rsem).start()
        pltpu.make_async_copy(wout_hbm.at[ctx_v], vp_v, rsem).start()

    def wait_rows(ctr_v, ctx_v, neg_v, rows_v, vc_v, vp_v, rsem):
        for j in range(4):
            sl = pl.ds(j * 80, 80)
            pltpu.make_async_copy(
                wout_hbm.at[neg_v.at[sl]], rows_v.at[sl], rsem).wait()
        pltpu.make_async_copy(win_hbm.at[ctr_v], vc_v, rsem).wait()
        pltpu.make_async_copy(wout_hbm.at[ctx_v], vp_v, rsem).wait()

    def fire_outs(c, oa_v, ob_v, oc_v, osem):
        b0 = base_of(c)
        pltpu.make_async_copy(
            oa_v, outa_hbm.at[pl.ds(b0, CH_)], osem).start()
        pltpu.make_async_copy(
            ob_v, outb_hbm.at[pl.ds(b0 // 4, 4)], osem).start()
        pltpu.make_async_copy(
            oc_v, outc_hbm.at[pl.ds(b0 // CH_, 1)], osem).start()

    def wait_outs(oa_v, ob_v, oc_v, osem):
        pltpu.make_async_copy(oa_v, outa_hbm.at[pl.ds(0, CH_)], osem).wait()
        pltpu.make_async_copy(ob_v, outb_hbm.at[pl.ds(0, 4)], osem).wait()
        pltpu.make_async_copy(oc_v, outc_hbm.at[pl.ds(0, 1)], osem).wait()

    def _rot(accs, j, t):
        out = list(accs)
        out[j % 4] = out[j % 4] + t
        return tuple(out)

    def _merge(accs):
        return (accs[0] + accs[1]) + (accs[2] + accs[3])

    def compute(rows_v, vc_v, vp_v, oa_v, ob_v, oc_v):
        def pass1(b, _):
            rowv = b * K_ + lanes

            def blk(blki, accs):
                vcb = vc_v[b, pl.ds(blki * 16, 16)]
                bs = blki * 16
                for j in range(16):
                    cv = vcb.at[jnp.full((16,), j, jnp.int32)].get(
                        mode="promise_in_bounds")
                    vals = plsc.load_gather(rows_v, [rowv, _i32x16(bs + j)])
                    accs = _rot(accs, j, vals * cv)
                return accs

            accs = (jnp.zeros((16,), jnp.float32),) * 4
            for mi, (lo, hi) in enumerate(SEGS_):
                accs = lax.fori_loop(lo // 16, hi // 16, blk, accs)
                oa_v[b, mi, :] = _merge(accs)
            return 0

        lax.fori_loop(0, CH_, pass1, 0)

        def pass2(sub, _):
            bvec = sub * 4 + lanes // 4
            rowv = bvec * K_ + 16 + (lanes % 4)

            def blk(blki, accs):
                bs = blki * 16
                for j in range(16):
                    cols = _i32x16(bs + j)
                    vals = plsc.load_gather(rows_v, [rowv, cols])
                    cv = plsc.load_gather(vc_v, [bvec, cols])
                    accs = _rot(accs, j, vals * cv)
                return accs

            accs = (jnp.zeros((16,), jnp.float32),) * 4
            for mi, (lo, hi) in enumerate(SEGS_):
                accs = lax.fori_loop(lo // 16, hi // 16, blk, accs)
                ob_v[sub, mi, :] = _merge(accs)
            return 0

        lax.fori_loop(0, 4, pass2, 0)

        def blk3(blki, accs):
            bs = blki * 16
            for j in range(16):
                cols = _i32x16(bs + j)
                pv = plsc.load_gather(vp_v, [lanes, cols])
                cv = plsc.load_gather(vc_v, [lanes, cols])
                accs = _rot(accs, j, pv * cv)
            return accs

        accs = (jnp.zeros((16,), jnp.float32),) * 4
        for mi, (lo, hi) in enumerate(SEGS_):
            accs = lax.fori_loop(lo // 16, hi // 16, blk3, accs)
            oc_v[0, mi, :] = _merge(accs)

    bufs0 = (ctr0, ctx0, neg0, rows0, vc0, vp0)
    bufs1 = (ctr1, ctx1, neg1, rows1, vc1, vp1)

    stage_idx(0, ctr0, ctx0, neg0, isem0)
    wait_idx(ctr0, ctx0, neg0, isem0)
    fire_rows(*bufs0, rsem0)
    stage_idx(1, ctr1, ctx1, neg1, isem1)

    def body(i, _):
        c0 = 2 * i
        wait_idx(ctr1, ctx1, neg1, isem1)
        fire_rows(*bufs1, rsem1)
        wait_rows(*bufs0, rsem0)

        @pl.when(i < NCHUNK_ // 2 - 1)
        def _():
            stage_idx(c0 + 2, ctr0, ctx0, neg0, isem0)

        @pl.when(i > 0)
        def _():
            wait_outs(oa0, ob0, oc0, osem0)

        compute(rows0, vc0, vp0, oa0, ob0, oc0)
        fire_outs(c0, oa0, ob0, oc0, osem0)

        @pl.when(i < NCHUNK_ // 2 - 1)
        def _():
            wait_idx(ctr0, ctx0, neg0, isem0)
            fire_rows(*bufs0, rsem0)

        wait_rows(*bufs1, rsem1)

        @pl.when(i < NCHUNK_ // 2 - 1)
        def _():
            stage_idx(c0 + 3, ctr1, ctx1, neg1, isem1)

        @pl.when(i > 0)
        def _():
            wait_outs(oa1, ob1, oc1, osem1)

        compute(rows1, vc1, vp1, oa1, ob1, oc1)
        fire_outs(c0 + 1, oa1, ob1, oc1, osem1)
        return 0

    lax.fori_loop(0, NCHUNK_ // 2, body, 0)
    wait_outs(oa0, ob0, oc0, osem0)
    wait_outs(oa1, ob1, oc1, osem1)


def _sc_scores(centers, contexts, negflat, w_in, w_out):
    mesh = plsc.VectorSubcoreMesh(core_axis_name="c", subcore_axis_name="s")
    f32 = jnp.float32
    i32 = jnp.int32
    bufset = [
        pltpu.VMEM((CH_,), i32),
        pltpu.VMEM((CH_,), i32),
        pltpu.VMEM((CH_ * K_,), i32),
        pltpu.VMEM((CH_ * K_, D_), f32),
        pltpu.VMEM((CH_, D_), f32),
        pltpu.VMEM((CH_, D_), f32),
    ]
    outset = [
        pltpu.VMEM((CH_, 4, 16), f32),
        pltpu.VMEM((4, 4, 16), f32),
        pltpu.VMEM((1, 4, 16), f32),
    ]
    kern = functools.partial(
        pl.kernel,
        out_type=(
            jax.ShapeDtypeStruct((B_, 4, 16), f32),
            jax.ShapeDtypeStruct((B_ // 4, 4, 16), f32),
            jax.ShapeDtypeStruct((B_ // CH_, 4, 16), f32),
        ),
        mesh=mesh,
        compiler_params=pltpu.CompilerParams(needs_layout_passes=False),
        scratch_types=bufset + bufset + outset + outset
        + [pltpu.SemaphoreType.DMA((6,))],
    )(_sc_body)
    return kern(centers, contexts, negflat, w_in, w_out)


def _tc_reduce(nega, negb, posc):
    def body(a_ref, b_ref, c_ref, o_ref):
        s = jnp.sum(jax.nn.log_sigmoid(-a_ref[...]))
        s = s + jnp.sum(jax.nn.log_sigmoid(-b_ref[...]))
        s = s + jnp.sum(jax.nn.log_sigmoid(c_ref[...]))
        o_ref[...] = jnp.broadcast_to(-s * (0.25 / B_), (1, 1))

    return pl.pallas_call(
        body,
        out_shape=jax.ShapeDtypeStruct((1, 1), jnp.float32),
    )(nega, negb, posc)


def kernel(centers, contexts, negatives, W_in, W_out):
    centers = centers.astype(jnp.int32)
    contexts = contexts.astype(jnp.int32)
    negflat = negatives.astype(jnp.int32).reshape(B_ * K_)
    nega, negb, posc = _sc_scores(centers, contexts, negflat, W_in, W_out)
    loss = _tc_reduce(
        nega.reshape(B_ * 64 // 128, 128),
        negb.reshape(B_ * 16 // 128, 128),
        posc.reshape(B_ * 4 // 128, 128),
    )
    return loss.reshape(())
